# Initial kernel scaffold; baseline (speedup 1.0000x reference)
import jax
import jax.numpy as jnp
from jax import lax
from jax.experimental import pallas as pl
from jax.experimental.pallas import tpu as pltpu, tpu_sc as plsc

_N = 100000
_K = 16
_C = 3
_NC = 2
_NS = 16
_NW = _NC * _NS
_NP = 102400
_PTS = _NP // _NW
_SUB = 800
_NSUB = _PTS // _SUB
_GRP = _SUB // 16


def _body(tab_hbm, idxT_hbm, out_hbm, part_hbm, tab_v, idx_v, acc_v, tot_v):
    wid = lax.axis_index("s") * _NC + lax.axis_index("c")
    base = wid * _PTS

    def zero(g, _):
        acc_v[pl.ds(g * 16, 16)] = jnp.zeros((16,), jnp.float32)
        return 0

    lax.fori_loop(0, _PTS // 16, zero, 0)

    for c in range(_C):
        pltpu.sync_copy(tab_hbm.at[c], tab_v)
        for sb in range(_NSUB):
            pltpu.sync_copy(
                idxT_hbm.at[:, pl.ds(base + sb * _SUB, _SUB)], idx_v)

            def grp(g, _, sb=sb):
                off = sb * _SUB + g * 16
                center = tab_v[pl.ds(base + off, 16)]
                a = acc_v[pl.ds(off, 16)]
                for k in range(1, _K):
                    idx = idx_v[k, pl.ds(g * 16, 16)]
                    nb = plsc.load_gather(tab_v, [idx])
                    a = a + jnp.abs(center - nb)
                acc_v[pl.ds(off, 16)] = a
                return 0

            lax.fori_loop(0, _GRP, grp, 0)

    lanes = lax.iota(jnp.int32, 16)

    def fin(g, tot):
        a = acc_v[pl.ds(g * 16, 16)] * jnp.float32(1.0 / (_K - 1))
        acc_v[pl.ds(g * 16, 16)] = a
        gp = base + g * 16 + lanes
        return tot + jnp.where(gp < _N, a, jnp.float32(0.0))

    tot = lax.fori_loop(0, _PTS // 16, fin, jnp.zeros((16,), jnp.float32))
    tot_v[...] = tot
    pltpu.sync_copy(tot_v, part_hbm.at[wid])
    pltpu.sync_copy(acc_v, out_hbm.at[pl.ds(base, _PTS)])


@jax.jit
def kernel(pred_flow, NN_idx):
    bs, n, c = pred_flow.shape
    flow = pred_flow.reshape(n, c)
    tab = jnp.zeros((_C, _NP), jnp.float32).at[:, :n].set(flow.T)
    idxT = jnp.zeros((_K, _NP), jnp.int32).at[:, :n].set(
        NN_idx.reshape(n, _K).T)

    f = pl.kernel(
        _body,
        out_type=(
            jax.ShapeDtypeStruct((_NP,), jnp.float32),
            jax.ShapeDtypeStruct((_NW, 16), jnp.float32),
        ),
        mesh=plsc.VectorSubcoreMesh(
            core_axis_name="c", subcore_axis_name="s"),
        scratch_types=[
            pltpu.VMEM((_NP,), jnp.float32),
            pltpu.VMEM((_K, _SUB), jnp.int32),
            pltpu.VMEM((_PTS,), jnp.float32),
            pltpu.VMEM((16,), jnp.float32),
        ],
    )
    per_point_pad, partials = f(tab, idxT)
    loss = jnp.sum(partials) / jnp.float32(n)
    per_point = per_point_pad[:n].reshape(bs, n)
    return (loss, per_point)


# speedup vs baseline: 44.8342x; 44.8342x over previous
import jax
import jax.numpy as jnp
from jax import lax
from jax.experimental import pallas as pl
from jax.experimental.pallas import tpu as pltpu, tpu_sc as plsc

_N = 100000
_K = 16
_C = 3
_NC = 2
_NS = 16
_NW = _NC * _NS
_NP = 102400
_PTS = _NP // _NW
_SUB = 640
_NSUB = _PTS // _SUB
_GRP = _SUB // 16


def _body(tab_hbm, idxT_hbm, out_hbm, part_hbm, tab_v, idx_v, acc_v, tot_v):
    wid = lax.axis_index("s") * _NC + lax.axis_index("c")
    base = wid * _PTS

    def zero(g, _):
        acc_v[pl.ds(g * 16, 16)] = jnp.zeros((16,), jnp.float32)
        return 0

    lax.fori_loop(0, _PTS // 16, zero, 0)

    for c in range(_C):
        pltpu.sync_copy(tab_hbm.at[pl.ds(c * _NP, _NP)], tab_v)
        for sb in range(_NSUB):
            pltpu.sync_copy(
                idxT_hbm.at[:, pl.ds(base + sb * _SUB, _SUB)], idx_v)

            def grp(g, _, sb=sb):
                off = sb * _SUB + g * 16
                center = tab_v[pl.ds(base + off, 16)]
                a = acc_v[pl.ds(off, 16)]
                for k in range(1, _K):
                    idx = idx_v[k, pl.ds(g * 16, 16)]
                    nb = plsc.load_gather(tab_v, [idx])
                    a = a + jnp.abs(center - nb)
                acc_v[pl.ds(off, 16)] = a
                return 0

            lax.fori_loop(0, _GRP, grp, 0)

    lanes = lax.iota(jnp.int32, 16)

    def fin(g, tot):
        a = acc_v[pl.ds(g * 16, 16)] * jnp.float32(1.0 / (_K - 1))
        acc_v[pl.ds(g * 16, 16)] = a
        gp = base + g * 16 + lanes
        return tot + jnp.where(gp < _N, a, jnp.float32(0.0))

    tot = lax.fori_loop(0, _PTS // 16, fin, jnp.zeros((16,), jnp.float32))
    tot_v[...] = tot
    pltpu.sync_copy(tot_v, part_hbm.at[wid])
    pltpu.sync_copy(acc_v, out_hbm.at[pl.ds(base, _PTS)])


@jax.jit
def kernel(pred_flow, NN_idx):
    bs, n, c = pred_flow.shape
    flow = pred_flow.reshape(n, c)
    tab = jnp.zeros((_C, _NP), jnp.float32).at[:, :n].set(
        flow.T).reshape(_C * _NP)
    idxT = jnp.zeros((_K, _NP), jnp.int32).at[:, :n].set(
        NN_idx.reshape(n, _K).T)

    f = pl.kernel(
        _body,
        out_type=(
            jax.ShapeDtypeStruct((_NP,), jnp.float32),
            jax.ShapeDtypeStruct((_NW, 16), jnp.float32),
        ),
        mesh=plsc.VectorSubcoreMesh(
            core_axis_name="c", subcore_axis_name="s"),
        compiler_params=pltpu.CompilerParams(needs_layout_passes=False),
        scratch_types=[
            pltpu.VMEM((_NP,), jnp.float32),
            pltpu.VMEM((_K, _SUB), jnp.int32),
            pltpu.VMEM((_PTS,), jnp.float32),
            pltpu.VMEM((16,), jnp.float32),
        ],
    )
    per_point_pad, partials = f(tab, idxT)
    loss = jnp.sum(partials) / jnp.float32(n)
    per_point = per_point_pad[:n].reshape(bs, n)
    return (loss, per_point)
